# Initial kernel scaffold; baseline (speedup 1.0000x reference)
import jax
import jax.numpy as jnp
from jax.experimental import pallas as pl


def kernel(x, s, emb, Wb, bb, W1, b1, W2, b2, W3, b3):
    raise NotImplementedError("write your pallas kernel here")


# speedup vs baseline: 9.9202x; 9.9202x over previous
import functools

import jax
import jax.numpy as jnp
from jax import lax
from jax.experimental import pallas as pl
from jax.experimental.pallas import tpu as pltpu
from jax.experimental.pallas import tpu_sc as plsc

B, N, K = 16, 2048, 5
C = 128
P = 12
T = 64


@functools.lru_cache(maxsize=None)
def _make_sc_gather(R, M, D, chunk=256):
    info = plsc.get_sparse_core_info()
    nw = info.num_cores * info.num_subcores
    per_w = M // nw
    n_chunks = per_w // chunk
    assert per_w % chunk == 0 and M % nw == 0
    mesh = plsc.VectorSubcoreMesh(core_axis_name="c", subcore_axis_name="s")

    @functools.partial(
        pl.kernel,
        out_type=jax.ShapeDtypeStruct((M, D), jnp.float32),
        mesh=mesh,
        scratch_types=[
            pltpu.VMEM((chunk,), jnp.int32),
            pltpu.VMEM((chunk, D), jnp.float32),
            pltpu.SemaphoreType.DMA,
        ],
    )
    def gather(table_hbm, idx_hbm, out_hbm, idx_v, rows_v, sem):
        wid = lax.axis_index("s") * info.num_cores + lax.axis_index("c")
        base = wid * per_w
        for i in range(n_chunks):
            off = base + i * chunk
            pltpu.sync_copy(idx_hbm.at[pl.ds(off, chunk)], idx_v)
            pltpu.async_copy(table_hbm.at[idx_v], rows_v, sem).wait()
            pltpu.sync_copy(rows_v, out_hbm.at[pl.ds(off, chunk)])

    return gather


def _sc_gather(table, idx):
    return _make_sc_gather(table.shape[0], idx.shape[0], table.shape[1])(
        table, idx)


def _tc_bilinear(e, props, Wcat, bb, blk=512):
    M = e.shape[0]

    def body(e_ref, p_ref, w_ref, b_ref, o_ref):
        eW = jnp.dot(e_ref[...], w_ref[...],
                     preferred_element_type=jnp.float32)
        acc = jnp.broadcast_to(b_ref[...], (blk, C))
        for k in range(P):
            acc = acc + p_ref[:, k:k + 1] * eW[:, k * C:(k + 1) * C]
        o_ref[...] = acc

    return pl.pallas_call(
        body,
        grid=(M // blk,),
        in_specs=[
            pl.BlockSpec((blk, C), lambda i: (i, 0)),
            pl.BlockSpec((blk, P), lambda i: (i, 0)),
            pl.BlockSpec((C, P * C), lambda i: (0, 0)),
            pl.BlockSpec((1, C), lambda i: (0, 0)),
        ],
        out_specs=pl.BlockSpec((blk, C), lambda i: (i, 0)),
        out_shape=jax.ShapeDtypeStruct((M, C), jnp.float32),
    )(e, props, Wcat, bb.reshape(1, C))


def _tc_iconv(g, W3d, b, act, blk=512):
    M, _, _ = g.shape
    Cout = W3d.shape[2]

    def body(g_ref, w_ref, b_ref, o_ref):
        acc = jnp.broadcast_to(b_ref[...], (blk, Cout))
        for k in range(K):
            acc = acc + jnp.dot(g_ref[:, k, :], w_ref[k],
                                preferred_element_type=jnp.float32)
        if act == "lrelu":
            acc = jnp.where(acc >= 0, acc, 0.01 * acc)
        elif act == "lsm":
            m = jnp.max(acc, axis=1, keepdims=True)
            acc = acc - m
            acc = acc - jnp.log(jnp.sum(jnp.exp(acc), axis=1, keepdims=True))
        o_ref[...] = acc

    return pl.pallas_call(
        body,
        grid=(M // blk,),
        in_specs=[
            pl.BlockSpec((blk, K, C), lambda i: (i, 0, 0)),
            pl.BlockSpec((K, C, Cout), lambda i: (0, 0, 0)),
            pl.BlockSpec((1, Cout), lambda i: (0, 0)),
        ],
        out_specs=pl.BlockSpec((blk, Cout), lambda i: (i, 0)),
        out_shape=jax.ShapeDtypeStruct((M, Cout), jnp.float32),
    )(g, W3d, b.reshape(1, Cout))


def kernel(x, s, emb, Wb, bb, W1, b1, W2, b2, W3, b3):
    ids = x[:, :, 0].reshape(-1).astype(jnp.int32)
    props = x[:, :, 1:].astype(jnp.float32).reshape(B * N, P)

    e = _sc_gather(emb, ids)

    Wcat = Wb.transpose(1, 2, 0).reshape(C, P * C)
    h = _tc_bilinear(e, props, Wcat, bb)

    offs = (jnp.arange(B, dtype=jnp.int32) * N)[:, None, None]
    sflat = (s.astype(jnp.int32) + offs).reshape(-1)

    for (W, b, act) in ((W1, b1, "lrelu"), (W2, b2, "lrelu"), (W3, b3, "lsm")):
        g = _sc_gather(h, sflat)
        Cout = W.shape[1]
        h = _tc_iconv(g.reshape(B * N, K, C), W.reshape(K, C, Cout), b, act)

    return jnp.transpose(h.reshape(B, N, T), (0, 2, 1))


# speedup vs baseline: 10.5231x; 1.0608x over previous
import functools

import jax
import jax.numpy as jnp
from jax import lax
from jax.experimental import pallas as pl
from jax.experimental.pallas import tpu as pltpu
from jax.experimental.pallas import tpu_sc as plsc

B, N, K = 16, 2048, 5
C = 128
P = 12
T = 64


@functools.lru_cache(maxsize=None)
def _make_sc_gather(R, M, D, chunk=320):
    info = plsc.get_sparse_core_info()
    nw = info.num_cores * info.num_subcores
    per_w = M // nw
    if per_w % chunk != 0:
        chunk = 256
    n_chunks = per_w // chunk
    assert per_w % chunk == 0 and M % nw == 0
    mesh = plsc.VectorSubcoreMesh(core_axis_name="c", subcore_axis_name="s")

    @functools.partial(
        pl.kernel,
        out_type=jax.ShapeDtypeStruct((M, D), jnp.float32),
        mesh=mesh,
        scratch_types=[
            pltpu.VMEM((per_w,), jnp.int32),
            pltpu.VMEM((2, chunk, D), jnp.float32),
            pltpu.SemaphoreType.DMA,
            pltpu.SemaphoreType.DMA,
            pltpu.SemaphoreType.DMA,
        ],
    )
    def gather(table_hbm, idx_hbm, out_hbm, idx_v, rows_v, sem_g, so0, so1):
        wid = lax.axis_index("s") * info.num_cores + lax.axis_index("c")
        base = wid * per_w
        pltpu.sync_copy(idx_hbm.at[pl.ds(base, per_w)], idx_v)
        sem_o = (so0, so1)
        out_h = [None, None]
        for i in range(n_chunks):
            b = i % 2
            if out_h[b] is not None:
                out_h[b].wait()
            pltpu.async_copy(
                table_hbm.at[idx_v.at[pl.ds(i * chunk, chunk)]],
                rows_v.at[b], sem_g).wait()
            out_h[b] = pltpu.async_copy(
                rows_v.at[b], out_hbm.at[pl.ds(base + i * chunk, chunk)],
                sem_o[b])
        for b in range(2):
            if out_h[b] is not None:
                out_h[b].wait()

    return gather


def _sc_gather(table, idx):
    return _make_sc_gather(table.shape[0], idx.shape[0], table.shape[1])(
        table, idx)


def _tc_bilinear(e, props, Wcat, bb, blk=512):
    M = e.shape[0]

    def body(e_ref, p_ref, w_ref, b_ref, o_ref):
        eW = jnp.dot(e_ref[...], w_ref[...],
                     preferred_element_type=jnp.float32)
        acc = jnp.broadcast_to(b_ref[...], (blk, C))
        for k in range(P):
            acc = acc + p_ref[:, k:k + 1] * eW[:, k * C:(k + 1) * C]
        o_ref[...] = acc

    return pl.pallas_call(
        body,
        grid=(M // blk,),
        in_specs=[
            pl.BlockSpec((blk, C), lambda i: (i, 0)),
            pl.BlockSpec((blk, P), lambda i: (i, 0)),
            pl.BlockSpec((C, P * C), lambda i: (0, 0)),
            pl.BlockSpec((1, C), lambda i: (0, 0)),
        ],
        out_specs=pl.BlockSpec((blk, C), lambda i: (i, 0)),
        out_shape=jax.ShapeDtypeStruct((M, C), jnp.float32),
    )(e, props, Wcat, bb.reshape(1, C))


def _tc_iconv(g, W3d, b, act, blk=512):
    M, _, _ = g.shape
    Cout = W3d.shape[2]

    def body(g_ref, w_ref, b_ref, o_ref):
        acc = jnp.broadcast_to(b_ref[...], (blk, Cout))
        for k in range(K):
            acc = acc + jnp.dot(g_ref[:, k, :], w_ref[k],
                                preferred_element_type=jnp.float32)
        if act == "lrelu":
            acc = jnp.where(acc >= 0, acc, 0.01 * acc)
        elif act == "lsm":
            m = jnp.max(acc, axis=1, keepdims=True)
            acc = acc - m
            acc = acc - jnp.log(jnp.sum(jnp.exp(acc), axis=1, keepdims=True))
        o_ref[...] = acc

    return pl.pallas_call(
        body,
        grid=(M // blk,),
        in_specs=[
            pl.BlockSpec((blk, K, C), lambda i: (i, 0, 0)),
            pl.BlockSpec((K, C, Cout), lambda i: (0, 0, 0)),
            pl.BlockSpec((1, Cout), lambda i: (0, 0)),
        ],
        out_specs=pl.BlockSpec((blk, Cout), lambda i: (i, 0)),
        out_shape=jax.ShapeDtypeStruct((M, Cout), jnp.float32),
    )(g, W3d, b.reshape(1, Cout))


def kernel(x, s, emb, Wb, bb, W1, b1, W2, b2, W3, b3):
    ids = x[:, :, 0].reshape(-1).astype(jnp.int32)
    props = x[:, :, 1:].astype(jnp.float32).reshape(B * N, P)

    e = _sc_gather(emb, ids)

    Wcat = Wb.transpose(1, 2, 0).reshape(C, P * C)
    h = _tc_bilinear(e, props, Wcat, bb)

    offs = (jnp.arange(B, dtype=jnp.int32) * N)[:, None, None]
    sflat = (s.astype(jnp.int32) + offs).reshape(-1)

    for (W, b, act) in ((W1, b1, "lrelu"), (W2, b2, "lrelu"), (W3, b3, "lsm")):
        g = _sc_gather(h, sflat)
        Cout = W.shape[1]
        h = _tc_iconv(g.reshape(B * N, K, C), W.reshape(K, C, Cout), b, act)

    return jnp.transpose(h.reshape(B, N, T), (0, 2, 1))


# speedup vs baseline: 18.8523x; 1.7915x over previous
import functools

import jax
import jax.numpy as jnp
from jax import lax
from jax.experimental import pallas as pl
from jax.experimental.pallas import tpu as pltpu
from jax.experimental.pallas import tpu_sc as plsc

B, N, K = 16, 2048, 5
C = 128
P = 12
T = 64


@functools.lru_cache(maxsize=None)
def _make_sc_gather(R, M, D, chunk=320):
    info = plsc.get_sparse_core_info()
    nw = info.num_cores * info.num_subcores
    per_w = M // nw
    if per_w % chunk != 0:
        chunk = 256
    n_chunks = per_w // chunk
    assert per_w % chunk == 0 and M % nw == 0
    mesh = plsc.VectorSubcoreMesh(core_axis_name="c", subcore_axis_name="s")

    @functools.partial(
        pl.kernel,
        out_type=jax.ShapeDtypeStruct((M, D), jnp.float32),
        mesh=mesh,
        scratch_types=[
            pltpu.VMEM((per_w,), jnp.int32),
            pltpu.VMEM((2, chunk, D), jnp.float32),
            pltpu.SemaphoreType.DMA,
            pltpu.SemaphoreType.DMA,
            pltpu.SemaphoreType.DMA,
        ],
    )
    def gather(table_hbm, idx_hbm, out_hbm, idx_v, rows_v, sem_g, so0, so1):
        wid = lax.axis_index("s") * info.num_cores + lax.axis_index("c")
        base = wid * per_w
        pltpu.sync_copy(idx_hbm.at[pl.ds(base, per_w)], idx_v)
        sem_o = (so0, so1)
        out_h = [None, None]
        for i in range(n_chunks):
            b = i % 2
            if out_h[b] is not None:
                out_h[b].wait()
            pltpu.async_copy(
                table_hbm.at[idx_v.at[pl.ds(i * chunk, chunk)]],
                rows_v.at[b], sem_g).wait()
            out_h[b] = pltpu.async_copy(
                rows_v.at[b], out_hbm.at[pl.ds(base + i * chunk, chunk)],
                sem_o[b])
        for b in range(2):
            if out_h[b] is not None:
                out_h[b].wait()

    return gather


def _sc_gather(table, idx):
    return _make_sc_gather(table.shape[0], idx.shape[0], table.shape[1])(
        table, idx)


def _tc_bilinear(e, props, Wcat, expand, bb, blk=512):
    M = e.shape[0]

    def body(e_ref, p_ref, w_ref, x_ref, b_ref, o_ref):
        eW = jnp.dot(e_ref[...], w_ref[...],
                     preferred_element_type=jnp.float32)
        pbig = jnp.dot(p_ref[...], x_ref[...],
                       preferred_element_type=jnp.float32)
        prod = pbig * eW
        acc = jnp.broadcast_to(b_ref[...], (blk, C))
        for k in range(P):
            acc = acc + prod[:, k * C:(k + 1) * C]
        o_ref[...] = acc

    return pl.pallas_call(
        body,
        grid=(M // blk,),
        in_specs=[
            pl.BlockSpec((blk, C), lambda i: (i, 0)),
            pl.BlockSpec((blk, P), lambda i: (i, 0)),
            pl.BlockSpec((C, P * C), lambda i: (0, 0)),
            pl.BlockSpec((P, P * C), lambda i: (0, 0)),
            pl.BlockSpec((1, C), lambda i: (0, 0)),
        ],
        out_specs=pl.BlockSpec((blk, C), lambda i: (i, 0)),
        out_shape=jax.ShapeDtypeStruct((M, C), jnp.float32),
    )(e, props, Wcat, expand, bb.reshape(1, C))


def _tc_iconv(g, W3d, b, act, blk=512):
    _, M, _ = g.shape
    Cout = W3d.shape[2]

    def body(g_ref, w_ref, b_ref, o_ref):
        acc = jnp.broadcast_to(b_ref[...], (blk, Cout))
        for k in range(K):
            acc = acc + jnp.dot(g_ref[k], w_ref[k],
                                preferred_element_type=jnp.float32)
        if act == "lrelu":
            acc = jnp.where(acc >= 0, acc, 0.01 * acc)
        elif act == "lsm":
            m = jnp.max(acc, axis=1, keepdims=True)
            acc = acc - m
            acc = acc - jnp.log(jnp.sum(jnp.exp(acc), axis=1, keepdims=True))
        o_ref[...] = acc

    return pl.pallas_call(
        body,
        grid=(M // blk,),
        in_specs=[
            pl.BlockSpec((K, blk, C), lambda i: (0, i, 0)),
            pl.BlockSpec((K, C, Cout), lambda i: (0, 0, 0)),
            pl.BlockSpec((1, Cout), lambda i: (0, 0)),
        ],
        out_specs=pl.BlockSpec((blk, Cout), lambda i: (i, 0)),
        out_shape=jax.ShapeDtypeStruct((M, Cout), jnp.float32),
    )(g, W3d, b.reshape(1, Cout))


def kernel(x, s, emb, Wb, bb, W1, b1, W2, b2, W3, b3):
    ids = x[:, :, 0].reshape(-1).astype(jnp.int32)
    props = x[:, :, 1:].astype(jnp.float32).reshape(B * N, P)

    e = _sc_gather(emb, ids)

    Wcat = Wb.transpose(1, 2, 0).reshape(C, P * C)
    expand = jnp.kron(jnp.eye(P, dtype=jnp.float32),
                      jnp.ones((1, C), dtype=jnp.float32))
    h = _tc_bilinear(e, props, Wcat, expand, bb)

    offs = (jnp.arange(B, dtype=jnp.int32) * N)[None, :, None]
    sflat = (s.astype(jnp.int32).transpose(2, 0, 1) + offs).reshape(-1)

    for (W, b, act) in ((W1, b1, "lrelu"), (W2, b2, "lrelu"), (W3, b3, "lsm")):
        g = _sc_gather(h, sflat)
        Cout = W.shape[1]
        h = _tc_iconv(g.reshape(K, B * N, C), W.reshape(K, C, Cout), b, act)

    return jnp.transpose(h.reshape(B, N, T), (0, 2, 1))


# speedup vs baseline: 19.1004x; 1.0132x over previous
import functools

import jax
import jax.numpy as jnp
from jax import lax
from jax.experimental import pallas as pl
from jax.experimental.pallas import tpu as pltpu
from jax.experimental.pallas import tpu_sc as plsc

B, N, K = 16, 2048, 5
C = 128
P = 12
T = 64


@functools.lru_cache(maxsize=None)
def _make_sc_gather(R, M, D, chunk=320):
    info = plsc.get_sparse_core_info()
    nw = info.num_cores * info.num_subcores
    per_w = M // nw
    if per_w % chunk != 0:
        chunk = 256
    n_chunks = per_w // chunk
    assert per_w % chunk == 0 and M % nw == 0
    mesh = plsc.VectorSubcoreMesh(core_axis_name="c", subcore_axis_name="s")

    @functools.partial(
        pl.kernel,
        out_type=jax.ShapeDtypeStruct((M, D), jnp.float32),
        mesh=mesh,
        scratch_types=[
            pltpu.VMEM((per_w,), jnp.int32),
            pltpu.VMEM((2, chunk, D), jnp.float32),
            pltpu.SemaphoreType.DMA,
            pltpu.SemaphoreType.DMA,
            pltpu.SemaphoreType.DMA,
            pltpu.SemaphoreType.DMA,
        ],
    )
    def gather(table_hbm, idx_hbm, out_hbm, idx_v, rows_v, sg0, sg1, so0, so1):
        wid = lax.axis_index("s") * info.num_cores + lax.axis_index("c")
        base = wid * per_w
        pltpu.sync_copy(idx_hbm.at[pl.ds(base, per_w)], idx_v)
        sem_g = (sg0, sg1)
        sem_o = (so0, so1)

        def start_gather(i):
            return pltpu.async_copy(
                table_hbm.at[idx_v.at[pl.ds(i * chunk, chunk)]],
                rows_v.at[i % 2], sem_g[i % 2])

        gat_h = [None, None]
        out_h = [None, None]
        gat_h[0] = start_gather(0)
        for i in range(n_chunks):
            b = i % 2
            nb = (i + 1) % 2
            if i + 1 < n_chunks:
                if out_h[nb] is not None:
                    out_h[nb].wait()
                gat_h[nb] = start_gather(i + 1)
            gat_h[b].wait()
            out_h[b] = pltpu.async_copy(
                rows_v.at[b], out_hbm.at[pl.ds(base + i * chunk, chunk)],
                sem_o[b])
        for b in range(2):
            if out_h[b] is not None:
                out_h[b].wait()

    return gather


def _sc_gather(table, idx):
    return _make_sc_gather(table.shape[0], idx.shape[0], table.shape[1])(
        table, idx)


def _tc_bilinear(e, props, Wcat, expand, bb, blk=512):
    M = e.shape[0]

    def body(e_ref, p_ref, w_ref, x_ref, b_ref, o_ref):
        eW = jnp.dot(e_ref[...], w_ref[...],
                     preferred_element_type=jnp.float32)
        pbig = jnp.dot(p_ref[...], x_ref[...],
                       preferred_element_type=jnp.float32)
        prod = pbig * eW
        acc = jnp.broadcast_to(b_ref[...], (blk, C))
        for k in range(P):
            acc = acc + prod[:, k * C:(k + 1) * C]
        o_ref[...] = acc

    return pl.pallas_call(
        body,
        grid=(M // blk,),
        in_specs=[
            pl.BlockSpec((blk, C), lambda i: (i, 0)),
            pl.BlockSpec((blk, P), lambda i: (i, 0)),
            pl.BlockSpec((C, P * C), lambda i: (0, 0)),
            pl.BlockSpec((P, P * C), lambda i: (0, 0)),
            pl.BlockSpec((1, C), lambda i: (0, 0)),
        ],
        out_specs=pl.BlockSpec((blk, C), lambda i: (i, 0)),
        out_shape=jax.ShapeDtypeStruct((M, C), jnp.float32),
    )(e, props, Wcat, expand, bb.reshape(1, C))


def _tc_iconv(g, W3d, b, act, blk=512):
    _, M, _ = g.shape
    Cout = W3d.shape[2]

    def body(g_ref, w_ref, b_ref, o_ref):
        acc = jnp.broadcast_to(b_ref[...], (blk, Cout))
        for k in range(K):
            acc = acc + jnp.dot(g_ref[k], w_ref[k],
                                preferred_element_type=jnp.float32)
        if act == "lrelu":
            acc = jnp.where(acc >= 0, acc, 0.01 * acc)
        elif act == "lsm":
            m = jnp.max(acc, axis=1, keepdims=True)
            acc = acc - m
            acc = acc - jnp.log(jnp.sum(jnp.exp(acc), axis=1, keepdims=True))
        o_ref[...] = acc

    return pl.pallas_call(
        body,
        grid=(M // blk,),
        in_specs=[
            pl.BlockSpec((K, blk, C), lambda i: (0, i, 0)),
            pl.BlockSpec((K, C, Cout), lambda i: (0, 0, 0)),
            pl.BlockSpec((1, Cout), lambda i: (0, 0)),
        ],
        out_specs=pl.BlockSpec((blk, Cout), lambda i: (i, 0)),
        out_shape=jax.ShapeDtypeStruct((M, Cout), jnp.float32),
    )(g, W3d, b.reshape(1, Cout))


def kernel(x, s, emb, Wb, bb, W1, b1, W2, b2, W3, b3):
    ids = x[:, :, 0].reshape(-1).astype(jnp.int32)
    props = x[:, :, 1:].astype(jnp.float32).reshape(B * N, P)

    e = _sc_gather(emb, ids)

    Wcat = Wb.transpose(1, 2, 0).reshape(C, P * C)
    expand = jnp.kron(jnp.eye(P, dtype=jnp.float32),
                      jnp.ones((1, C), dtype=jnp.float32))
    h = _tc_bilinear(e, props, Wcat, expand, bb)

    offs = (jnp.arange(B, dtype=jnp.int32) * N)[None, :, None]
    sflat = (s.astype(jnp.int32).transpose(2, 0, 1) + offs).reshape(-1)

    for (W, b, act) in ((W1, b1, "lrelu"), (W2, b2, "lrelu"), (W3, b3, "lsm")):
        g = _sc_gather(h, sflat)
        Cout = W.shape[1]
        h = _tc_iconv(g.reshape(K, B * N, C), W.reshape(K, C, Cout), b, act)

    return jnp.transpose(h.reshape(B, N, T), (0, 2, 1))
